# baseline (speedup 1.0000x reference)
import functools

import jax
import jax.numpy as jnp
from jax import lax
from jax.experimental import pallas as pl
from jax.experimental.pallas import tpu as pltpu
from jax.experimental.pallas import tpu_sc as plsc

BATCH = 16384
NUM_FIELDS = 26
VOCAB = 1000000
LANES = 16
NUM_CORES = 2
NUM_SUBCORES = 16
NUM_WORKERS = NUM_CORES * NUM_SUBCORES
BPW = BATCH // NUM_WORKERS
VPF = BPW // LANES
FLAT = NUM_FIELDS * BPW
CHUNK = 128
NCHUNK = FLAT // CHUNK


def _lr_body(idx_hbm, w_hbm, b_hbm, out_hbm, idx2_v, flat_v, rows_v, acc_v,
             bias_v, sem):
    wid = lax.axis_index("s") * NUM_CORES + lax.axis_index("c")
    base = wid * BPW

    pltpu.sync_copy(b_hbm, bias_v)
    pltpu.sync_copy(idx_hbm.at[:, pl.ds(base, BPW)], idx2_v)

    def _off_field(f, _):
        for j in range(VPF):
            dst = pl.ds(f * BPW + j * LANES, LANES)
            flat_v[dst] = idx2_v[f, pl.ds(j * LANES, LANES)] + f * VOCAB
        return 0

    lax.fori_loop(0, NUM_FIELDS, _off_field, 0)

    def _fire(c, _):
        sl = pl.ds(c * CHUNK, CHUNK)
        pltpu.async_copy(w_hbm.at[flat_v.at[sl]], rows_v.at[sl], sem)
        return 0

    lax.fori_loop(0, NCHUNK, _fire, 0)

    def _drain(c, _):
        sl = pl.ds(0, CHUNK)
        pltpu.make_async_copy(w_hbm.at[flat_v.at[sl]], rows_v.at[sl], sem).wait()
        return 0

    lax.fori_loop(0, NCHUNK, _drain, 0)

    bvec = bias_v[...]

    def _reduce(v, _):
        s = bvec
        for f in range(NUM_FIELDS):
            s = s + rows_v[pl.ds(f * BPW + v * LANES, LANES)]
        acc_v[pl.ds(v * LANES, LANES)] = s
        return 0

    lax.fori_loop(0, VPF, _reduce, 0)

    pltpu.sync_copy(acc_v, out_hbm.at[pl.ds(base, BPW)])


@jax.jit
def _lr_call(idx_t, w_flat, bias16):
    mesh = plsc.VectorSubcoreMesh(
        core_axis_name="c", subcore_axis_name="s",
        num_cores=NUM_CORES, num_subcores=NUM_SUBCORES,
    )
    return pl.kernel(
        _lr_body,
        out_type=jax.ShapeDtypeStruct((BATCH,), jnp.float32),
        mesh=mesh,
        scratch_types=[
            pltpu.VMEM((NUM_FIELDS, BPW), jnp.int32),
            pltpu.VMEM((FLAT,), jnp.int32),
            pltpu.VMEM((FLAT,), jnp.float32),
            pltpu.VMEM((BPW,), jnp.float32),
            pltpu.VMEM((LANES,), jnp.float32),
            pltpu.SemaphoreType.DMA,
        ],
    )(idx_t, w_flat, bias16)


def kernel(indices, W, bias):
    idx_t = indices.astype(jnp.int32).T
    w_flat = W.reshape(NUM_FIELDS * VOCAB)
    bias16 = jnp.broadcast_to(bias.astype(jnp.float32), (LANES,))
    out = _lr_call(idx_t, w_flat, bias16)
    return out.reshape(BATCH, 1)


# speedup vs baseline: 1.0010x; 1.0010x over previous
import functools

import jax
import jax.numpy as jnp
from jax import lax
from jax.experimental import pallas as pl
from jax.experimental.pallas import tpu as pltpu
from jax.experimental.pallas import tpu_sc as plsc

BATCH = 16384
NUM_FIELDS = 26
VOCAB = 1000000
LANES = 16
NUM_CORES = 2
NUM_SUBCORES = 16
NUM_WORKERS = NUM_CORES * NUM_SUBCORES
BPW = BATCH // NUM_WORKERS
VPF = BPW // LANES
FLAT = NUM_FIELDS * BPW
CHUNK = 128
NCHUNK = FLAT // CHUNK


def _lr_body(idx_hbm, w_hbm, b_hbm, out_hbm, idx2_v, rows_v, acc_v,
             bias_v, sem):
    wid = lax.axis_index("s") * NUM_CORES + lax.axis_index("c")
    base = wid * BPW

    pltpu.sync_copy(b_hbm, bias_v)
    pltpu.sync_copy(idx_hbm.at[:, pl.ds(base, BPW)], idx2_v)

    def _fire(c, _):
        f = c // VPF
        j = c % VPF
        iv = idx2_v[f, pl.ds(j * LANES, LANES)] + f * VOCAB
        pltpu.async_copy(w_hbm.at[iv], rows_v.at[pl.ds(c * LANES, LANES)], sem)
        return 0

    lax.fori_loop(0, NUM_FIELDS * VPF, _fire, 0)

    def _drain(c, _):
        iv = idx2_v[0, pl.ds(0, LANES)]
        pltpu.make_async_copy(w_hbm.at[iv], rows_v.at[pl.ds(0, LANES)], sem).wait()
        return 0

    lax.fori_loop(0, NUM_FIELDS * VPF, _drain, 0)

    bvec = bias_v[...]

    def _reduce(v, _):
        s = bvec
        for f in range(NUM_FIELDS):
            s = s + rows_v[pl.ds(f * BPW + v * LANES, LANES)]
        acc_v[pl.ds(v * LANES, LANES)] = s
        return 0

    lax.fori_loop(0, VPF, _reduce, 0)

    pltpu.sync_copy(acc_v, out_hbm.at[pl.ds(base, BPW)])


@jax.jit
def _lr_call(idx_t, w_flat, bias16):
    mesh = plsc.VectorSubcoreMesh(
        core_axis_name="c", subcore_axis_name="s",
        num_cores=NUM_CORES, num_subcores=NUM_SUBCORES,
    )
    return pl.kernel(
        _lr_body,
        out_type=jax.ShapeDtypeStruct((BATCH,), jnp.float32),
        mesh=mesh,
        scratch_types=[
            pltpu.VMEM((NUM_FIELDS, BPW), jnp.int32),
            pltpu.VMEM((FLAT,), jnp.float32),
            pltpu.VMEM((BPW,), jnp.float32),
            pltpu.VMEM((LANES,), jnp.float32),
            pltpu.SemaphoreType.DMA,
        ],
    )(idx_t, w_flat, bias16)


def kernel(indices, W, bias):
    idx_t = indices.astype(jnp.int32).T
    w_flat = W.reshape(NUM_FIELDS * VOCAB)
    bias16 = jnp.broadcast_to(bias.astype(jnp.float32), (LANES,))
    out = _lr_call(idx_t, w_flat, bias16)
    return out.reshape(BATCH, 1)


# speedup vs baseline: 8.5013x; 8.4929x over previous
import functools

import jax
import jax.numpy as jnp
from jax import lax
from jax.experimental import pallas as pl
from jax.experimental.pallas import tpu as pltpu
from jax.experimental.pallas import tpu_sc as plsc

BATCH = 16384
NUM_FIELDS = 26
VOCAB = 1000000
LANES = 16
NUM_CORES = 2
NUM_SUBCORES = 16
NUM_WORKERS = NUM_CORES * NUM_SUBCORES
BPW = BATCH // NUM_WORKERS
VPF = BPW // LANES
FLAT = NUM_FIELDS * BPW

CH = 31232
TAIL_START = CH * NUM_WORKERS
TAIL = VOCAB - TAIL_START


def _depad_body(w2_hbm, wlin_hbm, buf_v, tail_v, sem_i, sem_o):
    wid = lax.axis_index("s") * NUM_CORES + lax.axis_index("c")
    start = wid * CH

    pltpu.async_copy(w2_hbm.at[0, pl.ds(start, CH)], buf_v.at[pl.ds(0, CH)],
                     sem_i)

    def _body(f, _):
        pltpu.make_async_copy(
            w2_hbm.at[0, pl.ds(0, CH)], buf_v.at[pl.ds(0, CH)], sem_i).wait()

        @pl.when(f >= 1)
        def _():
            pltpu.make_async_copy(
                buf_v.at[pl.ds(0, CH)], wlin_hbm.at[pl.ds(0, CH)],
                sem_o).wait()

        @pl.when(f + 1 < NUM_FIELDS)
        def _():
            pltpu.async_copy(
                w2_hbm.at[f + 1, pl.ds(start, CH)],
                buf_v.at[pl.ds(((f + 1) % 2) * CH, CH)], sem_i)

        pltpu.async_copy(
            buf_v.at[pl.ds((f % 2) * CH, CH)],
            wlin_hbm.at[pl.ds(f * VOCAB + start, CH)], sem_o)
        return 0

    lax.fori_loop(0, NUM_FIELDS, _body, 0)
    pltpu.make_async_copy(
        buf_v.at[pl.ds(0, CH)], wlin_hbm.at[pl.ds(0, CH)], sem_o).wait()

    @pl.when(wid == NUM_WORKERS - 1)
    def _():
        def _tail(f, _):
            pltpu.sync_copy(w2_hbm.at[f, pl.ds(TAIL_START, TAIL)], tail_v)
            pltpu.sync_copy(
                tail_v, wlin_hbm.at[pl.ds(f * VOCAB + TAIL_START, TAIL)])
            return 0

        lax.fori_loop(0, NUM_FIELDS, _tail, 0)


def _lr_body(idx_hbm, w_hbm, b_hbm, out_hbm, idx2_v, rows_v, acc_v,
             bias_v, sem):
    wid = lax.axis_index("s") * NUM_CORES + lax.axis_index("c")
    base = wid * BPW

    pltpu.sync_copy(b_hbm, bias_v)
    pltpu.sync_copy(idx_hbm.at[:, pl.ds(base, BPW)], idx2_v)

    def _fire(c, _):
        f = c // VPF
        j = c % VPF
        iv = idx2_v[f, pl.ds(j * LANES, LANES)] + f * VOCAB
        pltpu.async_copy(w_hbm.at[iv], rows_v.at[c], sem)
        return 0

    lax.fori_loop(0, NUM_FIELDS * VPF, _fire, 0)

    def _drain(c, _):
        iv = idx2_v[0, pl.ds(0, LANES)]
        pltpu.make_async_copy(w_hbm.at[iv], rows_v.at[0], sem).wait()
        return 0

    lax.fori_loop(0, NUM_FIELDS * VPF, _drain, 0)

    bvec = bias_v[...]

    def _reduce(v, _):
        s = bvec
        for f in range(NUM_FIELDS):
            s = s + rows_v[f * VPF + v, :]
        acc_v[pl.ds(v * LANES, LANES)] = s
        return 0

    lax.fori_loop(0, VPF, _reduce, 0)

    pltpu.sync_copy(acc_v, out_hbm.at[pl.ds(base, BPW)])


@jax.jit
def _lr_call(idx_t, w2, bias16):
    mesh = plsc.VectorSubcoreMesh(
        core_axis_name="c", subcore_axis_name="s",
        num_cores=NUM_CORES, num_subcores=NUM_SUBCORES,
    )
    w_lin = pl.kernel(
        _depad_body,
        out_type=jax.ShapeDtypeStruct((NUM_FIELDS * VOCAB,), jnp.float32),
        mesh=mesh,
        scratch_types=[
            pltpu.VMEM((2 * CH,), jnp.float32),
            pltpu.VMEM((TAIL,), jnp.float32),
            pltpu.SemaphoreType.DMA,
            pltpu.SemaphoreType.DMA,
        ],
    )(w2)
    mesh2 = plsc.VectorSubcoreMesh(
        core_axis_name="c", subcore_axis_name="s",
        num_cores=NUM_CORES, num_subcores=NUM_SUBCORES,
    )
    return pl.kernel(
        _lr_body,
        out_type=jax.ShapeDtypeStruct((BATCH,), jnp.float32),
        mesh=mesh2,
        scratch_types=[
            pltpu.VMEM((NUM_FIELDS, BPW), jnp.int32),
            pltpu.VMEM((NUM_FIELDS * VPF, LANES), jnp.float32),
            pltpu.VMEM((BPW,), jnp.float32),
            pltpu.VMEM((LANES,), jnp.float32),
            pltpu.SemaphoreType.DMA,
        ],
    )(idx_t, w_lin, bias16)


def kernel(indices, W, bias):
    idx_t = indices.astype(jnp.int32).T
    w2 = W.reshape(NUM_FIELDS, VOCAB)
    bias16 = jnp.broadcast_to(bias.astype(jnp.float32), (LANES,))
    out = _lr_call(idx_t, w2, bias16)
    return out.reshape(BATCH, 1)


# speedup vs baseline: 8.8596x; 1.0421x over previous
import functools

import jax
import jax.numpy as jnp
from jax import lax
from jax.experimental import pallas as pl
from jax.experimental.pallas import tpu as pltpu
from jax.experimental.pallas import tpu_sc as plsc

BATCH = 16384
NUM_FIELDS = 26
VOCAB = 1000000
LANES = 16
NUM_CORES = 2
NUM_SUBCORES = 16
NUM_WORKERS = NUM_CORES * NUM_SUBCORES
BPW = BATCH // NUM_WORKERS
VPF = BPW // LANES
FLAT = NUM_FIELDS * BPW

CH = 31232
TAIL_START = CH * NUM_WORKERS
TAIL = VOCAB - TAIL_START


def _depad_body(w2_hbm, wlin_hbm, buf_v, tail_v, sem_i, sem_o):
    wid = lax.axis_index("s") * NUM_CORES + lax.axis_index("c")
    start = wid * CH

    pltpu.async_copy(w2_hbm.at[0, pl.ds(start, CH)],
                     buf_v.at[pl.ds(0, CH)], sem_i)

    def _body(f, _):
        pltpu.make_async_copy(
            w2_hbm.at[0, pl.ds(0, CH)], buf_v.at[pl.ds(0, CH)],
            sem_i).wait()

        @pl.when(f >= 1)
        def _():
            pltpu.make_async_copy(
                buf_v.at[pl.ds(0, CH)], wlin_hbm.at[pl.ds(0, CH)],
                sem_o).wait()

        @pl.when(f + 1 < NUM_FIELDS)
        def _():
            pltpu.async_copy(
                w2_hbm.at[f + 1, pl.ds(start, CH)],
                buf_v.at[pl.ds(((f + 1) % 2) * CH, CH)], sem_i)

        pltpu.async_copy(
            buf_v.at[pl.ds((f % 2) * CH, CH)],
            wlin_hbm.at[pl.ds(f * VOCAB + start, CH)], sem_o)
        return 0

    lax.fori_loop(0, NUM_FIELDS, _body, 0)
    pltpu.make_async_copy(
        buf_v.at[pl.ds(0, CH)], wlin_hbm.at[pl.ds(0, CH)], sem_o).wait()

    @pl.when(wid < NUM_FIELDS)
    def _():
        pltpu.sync_copy(w2_hbm.at[wid, pl.ds(TAIL_START, TAIL)], tail_v)
        pltpu.sync_copy(
            tail_v, wlin_hbm.at[pl.ds(wid * VOCAB + TAIL_START, TAIL)])


def _lr_body(idx_hbm, w_hbm, b_hbm, out_hbm, idx2_v, rows_v, acc_v,
             bias_v, sem):
    wid = lax.axis_index("s") * NUM_CORES + lax.axis_index("c")
    base = wid * BPW

    pltpu.sync_copy(b_hbm, bias_v)
    pltpu.sync_copy(idx_hbm.at[:, pl.ds(base, BPW)], idx2_v)

    def _fire(c, _):
        f = c // VPF
        j = c % VPF
        iv = idx2_v[f, pl.ds(j * LANES, LANES)] + f * VOCAB
        pltpu.async_copy(w_hbm.at[iv], rows_v.at[c], sem)
        return 0

    lax.fori_loop(0, NUM_FIELDS * VPF, _fire, 0)

    def _drain(c, _):
        iv = idx2_v[0, pl.ds(0, LANES)]
        pltpu.make_async_copy(w_hbm.at[iv], rows_v.at[0], sem).wait()
        return 0

    lax.fori_loop(0, NUM_FIELDS * VPF, _drain, 0)

    bvec = bias_v[...]

    def _reduce(v, _):
        s = bvec
        for f in range(NUM_FIELDS):
            s = s + rows_v[f * VPF + v, :]
        acc_v[pl.ds(v * LANES, LANES)] = s
        return 0

    lax.fori_loop(0, VPF, _reduce, 0)

    pltpu.sync_copy(acc_v, out_hbm.at[pl.ds(base, BPW)])


@jax.jit
def _lr_call(idx_t, w2, bias16):
    mesh = plsc.VectorSubcoreMesh(
        core_axis_name="c", subcore_axis_name="s",
        num_cores=NUM_CORES, num_subcores=NUM_SUBCORES,
    )
    w_lin = pl.kernel(
        _depad_body,
        out_type=jax.ShapeDtypeStruct((NUM_FIELDS * VOCAB,), jnp.float32),
        mesh=mesh,
        scratch_types=[
            pltpu.VMEM((2 * CH,), jnp.float32),
            pltpu.VMEM((TAIL,), jnp.float32),
            pltpu.SemaphoreType.DMA,
            pltpu.SemaphoreType.DMA,
        ],
    )(w2)
    mesh2 = plsc.VectorSubcoreMesh(
        core_axis_name="c", subcore_axis_name="s",
        num_cores=NUM_CORES, num_subcores=NUM_SUBCORES,
    )
    return pl.kernel(
        _lr_body,
        out_type=jax.ShapeDtypeStruct((BATCH,), jnp.float32),
        mesh=mesh2,
        scratch_types=[
            pltpu.VMEM((NUM_FIELDS, BPW), jnp.int32),
            pltpu.VMEM((NUM_FIELDS * VPF, LANES), jnp.float32),
            pltpu.VMEM((BPW,), jnp.float32),
            pltpu.VMEM((LANES,), jnp.float32),
            pltpu.SemaphoreType.DMA,
        ],
    )(idx_t, w_lin, bias16)


def kernel(indices, W, bias):
    idx_t = indices.astype(jnp.int32).T
    w2 = W.reshape(NUM_FIELDS, VOCAB)
    bias16 = jnp.broadcast_to(bias.astype(jnp.float32), (LANES,))
    out = _lr_call(idx_t, w2, bias16)
    return out.reshape(BATCH, 1)
